# baseline (speedup 1.0000x reference)
import functools

import jax
import jax.numpy as jnp
from jax import lax
from jax.experimental import pallas as pl
from jax.experimental.pallas import tpu as pltpu
from jax.experimental.pallas import tpu_sc as plsc

N = 10000
E = 320000
D = 128
D_OUT = 128

NC = 2
NS = 16
NW = NC * NS
CH = 128
CPT = 80
SB = 16
EPAD = NW * CPT * CH
NACC = 10240
RPS = NACC // NS
BN = 2000


def _sc_mesh():
    return plsc.VectorSubcoreMesh(core_axis_name="c", subcore_axis_name="s")


@functools.partial(
    pl.kernel,
    mesh=_sc_mesh(),
    out_type=jax.ShapeDtypeStruct((NC, NACC, D), jnp.float32),
    scratch_types=[
        pltpu.VMEM((CPT, CH), jnp.int32),
        pltpu.VMEM((CH, D), jnp.float32),
        pltpu.VMEM_SHARED((NACC, D), jnp.float32),
    ],
)
def _deg_call(col_hbm, zeros_hbm, ones_hbm, out_hbm, colv, onesv, acc):
    c = lax.axis_index("c")
    s = lax.axis_index("s")
    wid = s * NC + c
    pltpu.sync_copy(ones_hbm, onesv)
    for t in range(RPS // CH):
        pltpu.sync_copy(zeros_hbm, acc.at[pl.ds(s * RPS + t * CH, CH)])
    pltpu.sync_copy(col_hbm.at[pl.ds(wid * CPT, CPT)], colv)
    plsc.subcore_barrier()

    def body(j, carry):
        pltpu.sync_copy(onesv, acc.at[colv.at[j]], add=True)
        return carry

    lax.fori_loop(0, CPT, body, 0)
    plsc.subcore_barrier()
    pltpu.sync_copy(acc.at[pl.ds(s * RPS, RPS)],
                    out_hbm.at[c, pl.ds(s * RPS, RPS)])


@functools.partial(
    pl.kernel,
    mesh=_sc_mesh(),
    out_type=jax.ShapeDtypeStruct((NC, NACC, D), jnp.float32),
    scratch_types=[
        pltpu.VMEM((SB, CH), jnp.int32),
        pltpu.VMEM((SB, CH), jnp.int32),
        pltpu.VMEM((CH, D), jnp.float32),
        pltpu.VMEM((CH, D), jnp.float32),
        pltpu.VMEM_SHARED((NACC, D), jnp.float32),
        pltpu.SemaphoreType.DMA,
        pltpu.SemaphoreType.DMA,
        pltpu.SemaphoreType.DMA,
        pltpu.SemaphoreType.DMA,
    ],
)
def _hop_call(row_hbm, col_hbm, g_hbm, z_hbm, out_hbm,
              rowv, colv, buf0, buf1, acc, gsem0, gsem1, ssem0, ssem1):
    c = lax.axis_index("c")
    s = lax.axis_index("s")
    wid = s * NC + c
    pltpu.sync_copy(z_hbm, buf0)
    for t in range(RPS // CH):
        pltpu.sync_copy(buf0, acc.at[pl.ds(s * RPS + t * CH, CH)])
    plsc.subcore_barrier()

    nhalf = SB // 2

    def stage(ss, carry):
        base = wid * CPT + ss * SB
        pltpu.sync_copy(row_hbm.at[pl.ds(base, SB)], rowv)
        pltpu.sync_copy(col_hbm.at[pl.ds(base, SB)], colv)
        pltpu.async_copy(g_hbm.at[rowv.at[0]], buf0, gsem0)
        pltpu.async_copy(g_hbm.at[rowv.at[1]], buf1, gsem1)

        def body(jj, carry2):
            j0 = 2 * jj
            j1 = j0 + 1
            pltpu.make_async_copy(g_hbm.at[rowv.at[j0]], buf0, gsem0).wait()
            pltpu.async_copy(buf0, acc.at[colv.at[j0]], ssem0, add=True)
            pltpu.make_async_copy(g_hbm.at[rowv.at[j1]], buf1, gsem1).wait()
            pltpu.async_copy(buf1, acc.at[colv.at[j1]], ssem1, add=True)
            pltpu.make_async_copy(buf0, acc.at[colv.at[j0]], ssem0).wait()

            @pl.when(jj + 1 < nhalf)
            def _():
                pltpu.async_copy(g_hbm.at[rowv.at[j0 + 2]], buf0, gsem0)

            pltpu.make_async_copy(buf1, acc.at[colv.at[j1]], ssem1).wait()

            @pl.when(jj + 1 < nhalf)
            def _():
                pltpu.async_copy(g_hbm.at[rowv.at[j1 + 2]], buf1, gsem1)

            return carry2

        lax.fori_loop(0, nhalf, body, 0)
        return carry

    lax.fori_loop(0, CPT // SB, stage, 0)
    plsc.subcore_barrier()
    for t in range(RPS // CH):
        pltpu.sync_copy(acc.at[pl.ds(s * RPS + t * CH, CH)],
                        out_hbm.at[c, pl.ds(s * RPS + t * CH, CH)])


def _prep_body(dp_ref, x_ref, g0_ref, dinv_ref):
    deg = dp_ref[0, :, 0:1] + dp_ref[1, :, 0:1] + 1.0
    dinv = lax.rsqrt(deg)
    dinv_ref[...] = dinv
    g0_ref[...] = x_ref[...] * dinv


_prep_call = pl.pallas_call(
    _prep_body,
    grid=(N // BN,),
    in_specs=[
        pl.BlockSpec((2, BN, D), lambda i: (0, i, 0)),
        pl.BlockSpec((BN, D), lambda i: (i, 0)),
    ],
    out_specs=[
        pl.BlockSpec((BN, D), lambda i: (i, 0)),
        pl.BlockSpec((BN, 1), lambda i: (i, 0)),
    ],
    out_shape=[
        jax.ShapeDtypeStruct((N, D), jnp.float32),
        jax.ShapeDtypeStruct((N, 1), jnp.float32),
    ],
)


def _mid_body(p_ref, g0_ref, dinv_ref, g1_ref):
    d1 = dinv_ref[...]
    g1_ref[...] = (p_ref[0] + p_ref[1] + g0_ref[...]) * (d1 * d1)


_mid_call = pl.pallas_call(
    _mid_body,
    grid=(N // BN,),
    in_specs=[
        pl.BlockSpec((2, BN, D), lambda i: (0, i, 0)),
        pl.BlockSpec((BN, D), lambda i: (i, 0)),
        pl.BlockSpec((BN, 1), lambda i: (i, 0)),
    ],
    out_specs=pl.BlockSpec((BN, D), lambda i: (i, 0)),
    out_shape=jax.ShapeDtypeStruct((N, D), jnp.float32),
)


def _out_body(q_ref, g1_ref, dinv_ref, w_ref, b_ref, o_ref):
    h2 = (q_ref[0] + q_ref[1] + g1_ref[...]) * dinv_ref[...]
    o_ref[...] = jnp.dot(h2, w_ref[...],
                         preferred_element_type=jnp.float32) + b_ref[...]


_out_call = pl.pallas_call(
    _out_body,
    grid=(N // BN,),
    in_specs=[
        pl.BlockSpec((2, BN, D), lambda i: (0, i, 0)),
        pl.BlockSpec((BN, D), lambda i: (i, 0)),
        pl.BlockSpec((BN, 1), lambda i: (i, 0)),
        pl.BlockSpec((D, D_OUT), lambda i: (0, 0)),
        pl.BlockSpec((1, D_OUT), lambda i: (0, 0)),
    ],
    out_specs=pl.BlockSpec((BN, D_OUT), lambda i: (i, 0)),
    out_shape=jax.ShapeDtypeStruct((N, D_OUT), jnp.float32),
)


def kernel(x, edge_index, W, b):
    row = edge_index[0]
    col = edge_index[1]
    pad = EPAD - E
    ar = jnp.arange(pad, dtype=jnp.int32)
    pad_rows = (ar * 37) % N
    pad_cols = N + ar % (NACC - N)
    row2d = jnp.concatenate([row, pad_rows]).reshape(NW * CPT, CH)
    col2d = jnp.concatenate([col, pad_cols]).reshape(NW * CPT, CH)

    ones_rows = jnp.ones((CH, D), jnp.float32)
    z_rows = jnp.zeros((CH, D), jnp.float32)

    dp = _deg_call(col2d, z_rows, ones_rows)
    g0, dinv = _prep_call(dp, x)
    p = _hop_call(row2d, col2d, g0, z_rows)
    g1 = _mid_call(p, g0, dinv)
    q = _hop_call(row2d, col2d, g1, z_rows)
    out = _out_call(q, g1, dinv, W, jnp.reshape(b, (1, D_OUT)))
    return out


# speedup vs baseline: 1.1721x; 1.1721x over previous
import functools

import jax
import jax.numpy as jnp
from jax import lax
from jax.experimental import pallas as pl
from jax.experimental.pallas import tpu as pltpu
from jax.experimental.pallas import tpu_sc as plsc

N = 10000
E = 320000
D = 128
D_OUT = 128

NC = 2
NS = 16
NW = NC * NS
CH = 128
CPT = 80
SB = 16
EPAD = NW * CPT * CH
NACC = 10240
RPS = NACC // NS
BN = 2000


def _sc_mesh():
    return plsc.VectorSubcoreMesh(core_axis_name="c", subcore_axis_name="s")


@functools.partial(
    pl.kernel,
    mesh=_sc_mesh(),
    out_type=jax.ShapeDtypeStruct((NC, NACC, D), jnp.float32),
    scratch_types=[
        pltpu.VMEM((CPT, CH), jnp.int32),
        pltpu.VMEM((CH, D), jnp.float32),
        pltpu.VMEM_SHARED((NACC, D), jnp.float32),
    ],
)
def _deg_call(col_hbm, zeros_hbm, ones_hbm, out_hbm, colv, onesv, acc):
    c = lax.axis_index("c")
    s = lax.axis_index("s")
    wid = s * NC + c
    pltpu.sync_copy(ones_hbm, onesv)
    for t in range(RPS // CH):
        pltpu.sync_copy(zeros_hbm, acc.at[pl.ds(s * RPS + t * CH, CH)])
    pltpu.sync_copy(col_hbm.at[pl.ds(wid * CPT, CPT)], colv)
    plsc.subcore_barrier()

    def body(j, carry):
        pltpu.sync_copy(onesv, acc.at[colv.at[j]], add=True)
        return carry

    lax.fori_loop(0, CPT, body, 0)
    plsc.subcore_barrier()
    pltpu.sync_copy(acc.at[pl.ds(s * RPS, RPS)],
                    out_hbm.at[c, pl.ds(s * RPS, RPS)])


@functools.partial(
    pl.kernel,
    mesh=_sc_mesh(),
    out_type=jax.ShapeDtypeStruct((NC, NACC, D), jnp.float32),
    scratch_types=[
        pltpu.VMEM((SB, CH), jnp.int32),
        pltpu.VMEM((SB, CH), jnp.int32),
        pltpu.VMEM((CH, D), jnp.float32),
        pltpu.VMEM((CH, D), jnp.float32),
        pltpu.VMEM_SHARED((NACC, D), jnp.float32),
        pltpu.SemaphoreType.DMA,
        pltpu.SemaphoreType.DMA,
        pltpu.SemaphoreType.DMA,
        pltpu.SemaphoreType.DMA,
    ],
)
def _hop_call(row_hbm, col_hbm, g_hbm, z_hbm, out_hbm,
              rowv, colv, buf0, buf1, acc, gsem0, gsem1, ssem0, ssem1):
    c = lax.axis_index("c")
    s = lax.axis_index("s")
    wid = s * NC + c
    pltpu.sync_copy(z_hbm, buf0)
    for t in range(RPS // CH):
        pltpu.sync_copy(buf0, acc.at[pl.ds(s * RPS + t * CH, CH)])
    plsc.subcore_barrier()

    nhalf = SB // 2

    def stage(ss, carry):
        base = wid * CPT + ss * SB
        pltpu.sync_copy(row_hbm.at[pl.ds(base, SB)], rowv)
        pltpu.sync_copy(col_hbm.at[pl.ds(base, SB)], colv)
        pltpu.async_copy(g_hbm.at[rowv.at[0]], buf0, gsem0)

        def body(jj, carry2):
            j0 = 2 * jj
            j1 = j0 + 1
            pltpu.async_copy(g_hbm.at[rowv.at[j1]], buf1, gsem1)
            pltpu.make_async_copy(g_hbm.at[rowv.at[j0]], buf0, gsem0).wait()
            pltpu.sync_copy(buf0, acc.at[colv.at[j0]], add=True)

            @pl.when(jj + 1 < nhalf)
            def _():
                pltpu.async_copy(g_hbm.at[rowv.at[j0 + 2]], buf0, gsem0)

            pltpu.make_async_copy(g_hbm.at[rowv.at[j1]], buf1, gsem1).wait()
            pltpu.sync_copy(buf1, acc.at[colv.at[j1]], add=True)
            return carry2

        lax.fori_loop(0, nhalf, body, 0)
        return carry

    lax.fori_loop(0, CPT // SB, stage, 0)
    plsc.subcore_barrier()
    for t in range(RPS // CH):
        pltpu.sync_copy(acc.at[pl.ds(s * RPS + t * CH, CH)],
                        out_hbm.at[c, pl.ds(s * RPS + t * CH, CH)])


def _prep_body(dp_ref, x_ref, g0_ref, dinv_ref):
    deg = dp_ref[0, :, 0:1] + dp_ref[1, :, 0:1] + 1.0
    dinv = lax.rsqrt(deg)
    dinv_ref[...] = dinv
    g0_ref[...] = x_ref[...] * dinv


_prep_call = pl.pallas_call(
    _prep_body,
    grid=(N // BN,),
    in_specs=[
        pl.BlockSpec((2, BN, D), lambda i: (0, i, 0)),
        pl.BlockSpec((BN, D), lambda i: (i, 0)),
    ],
    out_specs=[
        pl.BlockSpec((BN, D), lambda i: (i, 0)),
        pl.BlockSpec((BN, 1), lambda i: (i, 0)),
    ],
    out_shape=[
        jax.ShapeDtypeStruct((N, D), jnp.float32),
        jax.ShapeDtypeStruct((N, 1), jnp.float32),
    ],
)


def _mid_body(p_ref, g0_ref, dinv_ref, g1_ref):
    d1 = dinv_ref[...]
    g1_ref[...] = (p_ref[0] + p_ref[1] + g0_ref[...]) * (d1 * d1)


_mid_call = pl.pallas_call(
    _mid_body,
    grid=(N // BN,),
    in_specs=[
        pl.BlockSpec((2, BN, D), lambda i: (0, i, 0)),
        pl.BlockSpec((BN, D), lambda i: (i, 0)),
        pl.BlockSpec((BN, 1), lambda i: (i, 0)),
    ],
    out_specs=pl.BlockSpec((BN, D), lambda i: (i, 0)),
    out_shape=jax.ShapeDtypeStruct((N, D), jnp.float32),
)


def _out_body(q_ref, g1_ref, dinv_ref, w_ref, b_ref, o_ref):
    h2 = (q_ref[0] + q_ref[1] + g1_ref[...]) * dinv_ref[...]
    o_ref[...] = jnp.dot(h2, w_ref[...],
                         preferred_element_type=jnp.float32) + b_ref[...]


_out_call = pl.pallas_call(
    _out_body,
    grid=(N // BN,),
    in_specs=[
        pl.BlockSpec((2, BN, D), lambda i: (0, i, 0)),
        pl.BlockSpec((BN, D), lambda i: (i, 0)),
        pl.BlockSpec((BN, 1), lambda i: (i, 0)),
        pl.BlockSpec((D, D_OUT), lambda i: (0, 0)),
        pl.BlockSpec((1, D_OUT), lambda i: (0, 0)),
    ],
    out_specs=pl.BlockSpec((BN, D_OUT), lambda i: (i, 0)),
    out_shape=jax.ShapeDtypeStruct((N, D_OUT), jnp.float32),
)


def kernel(x, edge_index, W, b):
    row = edge_index[0]
    col = edge_index[1]
    pad = EPAD - E
    ar = jnp.arange(pad, dtype=jnp.int32)
    pad_rows = (ar * 37) % N
    pad_cols = N + ar % (NACC - N)
    row2d = jnp.concatenate([row, pad_rows]).reshape(NW * CPT, CH)
    col2d = jnp.concatenate([col, pad_cols]).reshape(NW * CPT, CH)

    ones_rows = jnp.ones((CH, D), jnp.float32)
    z_rows = jnp.zeros((CH, D), jnp.float32)

    dp = _deg_call(col2d, z_rows, ones_rows)
    g0, dinv = _prep_call(dp, x)
    p = _hop_call(row2d, col2d, g0, z_rows)
    g1 = _mid_call(p, g0, dinv)
    q = _hop_call(row2d, col2d, g1, z_rows)
    out = _out_call(q, g1, dinv, W, jnp.reshape(b, (1, D_OUT)))
    return out


# speedup vs baseline: 1.2021x; 1.0256x over previous
import functools

import jax
import jax.numpy as jnp
from jax import lax
from jax.experimental import pallas as pl
from jax.experimental.pallas import tpu as pltpu
from jax.experimental.pallas import tpu_sc as plsc

N = 10000
E = 320000
D = 128
D_OUT = 128

NC = 2
NS = 16
NW = NC * NS
CH = 128
CPT = 80
SB = 16
EPAD = NW * CPT * CH
NACC = 10240
RPS = NACC // NS
BN = 2000


def _sc_mesh():
    return plsc.VectorSubcoreMesh(core_axis_name="c", subcore_axis_name="s")


@functools.partial(
    pl.kernel,
    mesh=_sc_mesh(),
    out_type=jax.ShapeDtypeStruct((NC, NACC, D), jnp.float32),
    scratch_types=[
        pltpu.VMEM((CPT, CH), jnp.int32),
        pltpu.VMEM((CH, D), jnp.float32),
        pltpu.VMEM_SHARED((NACC, D), jnp.float32),
    ],
)
def _deg_call(col_hbm, zeros_hbm, ones_hbm, out_hbm, colv, onesv, acc):
    c = lax.axis_index("c")
    s = lax.axis_index("s")
    wid = s * NC + c
    pltpu.sync_copy(ones_hbm, onesv)
    for t in range(RPS // CH):
        pltpu.sync_copy(zeros_hbm, acc.at[pl.ds(s * RPS + t * CH, CH)])
    pltpu.sync_copy(col_hbm.at[pl.ds(wid * CPT, CPT)], colv)
    plsc.subcore_barrier()

    def body(j, carry):
        pltpu.sync_copy(onesv, acc.at[colv.at[j]], add=True)
        return carry

    lax.fori_loop(0, CPT, body, 0)
    plsc.subcore_barrier()
    pltpu.sync_copy(acc.at[pl.ds(s * RPS, RPS)],
                    out_hbm.at[c, pl.ds(s * RPS, RPS)])


@functools.partial(
    pl.kernel,
    mesh=_sc_mesh(),
    out_type=jax.ShapeDtypeStruct((NC, NACC, D), jnp.float32),
    scratch_types=[
        pltpu.VMEM((2, SB, CH), jnp.int32),
        pltpu.VMEM((2, SB, CH), jnp.int32),
        pltpu.VMEM((CH, D), jnp.float32),
        pltpu.VMEM((CH, D), jnp.float32),
        pltpu.VMEM_SHARED((NACC, D), jnp.float32),
        pltpu.SemaphoreType.DMA,
        pltpu.SemaphoreType.DMA,
        pltpu.SemaphoreType.DMA,
        pltpu.SemaphoreType.DMA,
        pltpu.SemaphoreType.DMA,
    ],
)
def _hop_call(row_hbm, col_hbm, g_hbm, z_hbm, out_hbm,
              rowv, colv, buf0, buf1, acc, gsem0, gsem1, isem_r, isem_c, zsem):
    c = lax.axis_index("c")
    s = lax.axis_index("s")
    wid = s * NC + c
    nhalf = SB // 2
    nstg = CPT // SB

    pltpu.async_copy(row_hbm.at[pl.ds(wid * CPT, SB)], rowv.at[0], isem_r)
    pltpu.async_copy(col_hbm.at[pl.ds(wid * CPT, SB)], colv.at[0], isem_c)
    pltpu.sync_copy(z_hbm, buf0)
    for t in range(RPS // CH):
        pltpu.async_copy(buf0, acc.at[pl.ds(s * RPS + t * CH, CH)], zsem)
    for t in range(RPS // CH):
        pltpu.make_async_copy(buf0, acc.at[pl.ds(s * RPS + t * CH, CH)],
                              zsem).wait()
    pltpu.make_async_copy(row_hbm.at[pl.ds(wid * CPT, SB)], rowv.at[0],
                          isem_r).wait()
    pltpu.make_async_copy(col_hbm.at[pl.ds(wid * CPT, SB)], colv.at[0],
                          isem_c).wait()
    plsc.subcore_barrier()

    for ss in range(nstg):
        pb = ss % 2
        nb = 1 - pb
        nxt = wid * CPT + (ss + 1) * SB
        if ss + 1 < nstg:
            pltpu.async_copy(row_hbm.at[pl.ds(nxt, SB)], rowv.at[nb], isem_r)
            pltpu.async_copy(col_hbm.at[pl.ds(nxt, SB)], colv.at[nb], isem_c)

        pltpu.async_copy(g_hbm.at[rowv.at[pb, 0]], buf0, gsem0)

        def body(jj, carry2, pb=pb):
            j0 = 2 * jj
            j1 = j0 + 1
            pltpu.async_copy(g_hbm.at[rowv.at[pb, j1]], buf1, gsem1)
            pltpu.make_async_copy(g_hbm.at[rowv.at[pb, j0]], buf0,
                                  gsem0).wait()
            pltpu.sync_copy(buf0, acc.at[colv.at[pb, j0]], add=True)

            @pl.when(jj + 1 < nhalf)
            def _():
                pltpu.async_copy(g_hbm.at[rowv.at[pb, j0 + 2]], buf0, gsem0)

            pltpu.make_async_copy(g_hbm.at[rowv.at[pb, j1]], buf1,
                                  gsem1).wait()
            pltpu.sync_copy(buf1, acc.at[colv.at[pb, j1]], add=True)
            return carry2

        lax.fori_loop(0, nhalf, body, 0)
        if ss + 1 < nstg:
            pltpu.make_async_copy(row_hbm.at[pl.ds(nxt, SB)], rowv.at[nb],
                                  isem_r).wait()
            pltpu.make_async_copy(col_hbm.at[pl.ds(nxt, SB)], colv.at[nb],
                                  isem_c).wait()

    plsc.subcore_barrier()
    for t in range(RPS // CH):
        pltpu.async_copy(acc.at[pl.ds(s * RPS + t * CH, CH)],
                         out_hbm.at[c, pl.ds(s * RPS + t * CH, CH)], zsem)
    for t in range(RPS // CH):
        pltpu.make_async_copy(acc.at[pl.ds(s * RPS + t * CH, CH)],
                              out_hbm.at[c, pl.ds(s * RPS + t * CH, CH)],
                              zsem).wait()


def _prep_body(dp_ref, x_ref, g0_ref, dinv_ref):
    deg = dp_ref[0, :, 0:1] + dp_ref[1, :, 0:1] + 1.0
    dinv = lax.rsqrt(deg)
    dinv_ref[...] = dinv
    g0_ref[...] = x_ref[...] * dinv


_prep_call = pl.pallas_call(
    _prep_body,
    grid=(N // BN,),
    in_specs=[
        pl.BlockSpec((2, BN, D), lambda i: (0, i, 0)),
        pl.BlockSpec((BN, D), lambda i: (i, 0)),
    ],
    out_specs=[
        pl.BlockSpec((BN, D), lambda i: (i, 0)),
        pl.BlockSpec((BN, 1), lambda i: (i, 0)),
    ],
    out_shape=[
        jax.ShapeDtypeStruct((N, D), jnp.float32),
        jax.ShapeDtypeStruct((N, 1), jnp.float32),
    ],
)


def _mid_body(p_ref, g0_ref, dinv_ref, g1_ref):
    d1 = dinv_ref[...]
    g1_ref[...] = (p_ref[0] + p_ref[1] + g0_ref[...]) * (d1 * d1)


_mid_call = pl.pallas_call(
    _mid_body,
    grid=(N // BN,),
    in_specs=[
        pl.BlockSpec((2, BN, D), lambda i: (0, i, 0)),
        pl.BlockSpec((BN, D), lambda i: (i, 0)),
        pl.BlockSpec((BN, 1), lambda i: (i, 0)),
    ],
    out_specs=pl.BlockSpec((BN, D), lambda i: (i, 0)),
    out_shape=jax.ShapeDtypeStruct((N, D), jnp.float32),
)


def _out_body(q_ref, g1_ref, dinv_ref, w_ref, b_ref, o_ref):
    h2 = (q_ref[0] + q_ref[1] + g1_ref[...]) * dinv_ref[...]
    o_ref[...] = jnp.dot(h2, w_ref[...],
                         preferred_element_type=jnp.float32) + b_ref[...]


_out_call = pl.pallas_call(
    _out_body,
    grid=(N // BN,),
    in_specs=[
        pl.BlockSpec((2, BN, D), lambda i: (0, i, 0)),
        pl.BlockSpec((BN, D), lambda i: (i, 0)),
        pl.BlockSpec((BN, 1), lambda i: (i, 0)),
        pl.BlockSpec((D, D_OUT), lambda i: (0, 0)),
        pl.BlockSpec((1, D_OUT), lambda i: (0, 0)),
    ],
    out_specs=pl.BlockSpec((BN, D_OUT), lambda i: (i, 0)),
    out_shape=jax.ShapeDtypeStruct((N, D_OUT), jnp.float32),
)


def kernel(x, edge_index, W, b):
    row = edge_index[0]
    col = edge_index[1]
    pad = EPAD - E
    ar = jnp.arange(pad, dtype=jnp.int32)
    pad_rows = (ar * 37) % N
    pad_cols = N + ar % (NACC - N)
    row2d = jnp.concatenate([row, pad_rows]).reshape(NW * CPT, CH)
    col2d = jnp.concatenate([col, pad_cols]).reshape(NW * CPT, CH)

    ones_rows = jnp.ones((CH, D), jnp.float32)
    z_rows = jnp.zeros((CH, D), jnp.float32)

    dp = _deg_call(col2d, z_rows, ones_rows)
    g0, dinv = _prep_call(dp, x)
    p = _hop_call(row2d, col2d, g0, z_rows)
    g1 = _mid_call(p, g0, dinv)
    q = _hop_call(row2d, col2d, g1, z_rows)
    out = _out_call(q, g1, dinv, W, jnp.reshape(b, (1, D_OUT)))
    return out


# speedup vs baseline: 1.2055x; 1.0028x over previous
import functools

import jax
import jax.numpy as jnp
from jax import lax
from jax.experimental import pallas as pl
from jax.experimental.pallas import tpu as pltpu
from jax.experimental.pallas import tpu_sc as plsc

N = 10000
E = 320000
D = 128
D_OUT = 128

NC = 2
NS = 16
NW = NC * NS
CH = 128
CPT = 80
SB = 16
EPAD = NW * CPT * CH
NACC = 10240
RPS = NACC // NS
BN = 2000


def _sc_mesh():
    return plsc.VectorSubcoreMesh(core_axis_name="c", subcore_axis_name="s")


@functools.partial(
    pl.kernel,
    mesh=_sc_mesh(),
    out_type=jax.ShapeDtypeStruct((NC, NACC, D), jnp.float32),
    scratch_types=[
        pltpu.VMEM((CPT, CH), jnp.int32),
        pltpu.VMEM((CH, D), jnp.float32),
        pltpu.VMEM_SHARED((NACC, D), jnp.float32),
        pltpu.SemaphoreType.DMA,
    ],
)
def _deg_call(col_hbm, zeros_hbm, ones_hbm, out_hbm, colv, onesv, acc, ssem):
    c = lax.axis_index("c")
    s = lax.axis_index("s")
    wid = s * NC + c
    pltpu.sync_copy(ones_hbm, onesv)
    for t in range(RPS // CH):
        pltpu.sync_copy(zeros_hbm, acc.at[pl.ds(s * RPS + t * CH, CH)])
    pltpu.sync_copy(col_hbm.at[pl.ds(wid * CPT, CPT)], colv)
    plsc.subcore_barrier()

    WIN = 4

    def body(j, carry):
        pltpu.async_copy(onesv, acc.at[colv.at[j]], ssem, add=True)

        @pl.when(j >= WIN)
        def _():
            pltpu.make_async_copy(onesv, acc.at[colv.at[j - WIN]], ssem).wait()

        return carry

    lax.fori_loop(0, CPT, body, 0)
    for t in range(WIN):
        pltpu.make_async_copy(onesv, acc.at[colv.at[CPT - WIN + t]],
                              ssem).wait()
    plsc.subcore_barrier()
    pltpu.sync_copy(acc.at[pl.ds(s * RPS, RPS)],
                    out_hbm.at[c, pl.ds(s * RPS, RPS)])


@functools.partial(
    pl.kernel,
    mesh=_sc_mesh(),
    out_type=jax.ShapeDtypeStruct((NC, NACC, D), jnp.float32),
    scratch_types=[
        pltpu.VMEM((2, SB, CH), jnp.int32),
        pltpu.VMEM((2, SB, CH), jnp.int32),
        pltpu.VMEM((CH, D), jnp.float32),
        pltpu.VMEM((CH, D), jnp.float32),
        pltpu.VMEM_SHARED((NACC, D), jnp.float32),
        pltpu.SemaphoreType.DMA,
        pltpu.SemaphoreType.DMA,
        pltpu.SemaphoreType.DMA,
        pltpu.SemaphoreType.DMA,
        pltpu.SemaphoreType.DMA,
    ],
)
def _hop_call(row_hbm, col_hbm, g_hbm, z_hbm, out_hbm,
              rowv, colv, buf0, buf1, acc, gsem0, gsem1, isem_r, isem_c, zsem):
    c = lax.axis_index("c")
    s = lax.axis_index("s")
    wid = s * NC + c
    nhalf = SB // 2
    nstg = CPT // SB

    pltpu.async_copy(row_hbm.at[pl.ds(wid * CPT, SB)], rowv.at[0], isem_r)
    pltpu.async_copy(col_hbm.at[pl.ds(wid * CPT, SB)], colv.at[0], isem_c)
    pltpu.sync_copy(z_hbm, buf0)
    for t in range(RPS // CH):
        pltpu.async_copy(buf0, acc.at[pl.ds(s * RPS + t * CH, CH)], zsem)
    for t in range(RPS // CH):
        pltpu.make_async_copy(buf0, acc.at[pl.ds(s * RPS + t * CH, CH)],
                              zsem).wait()
    pltpu.make_async_copy(row_hbm.at[pl.ds(wid * CPT, SB)], rowv.at[0],
                          isem_r).wait()
    pltpu.make_async_copy(col_hbm.at[pl.ds(wid * CPT, SB)], colv.at[0],
                          isem_c).wait()
    plsc.subcore_barrier()

    for ss in range(nstg):
        pb = ss % 2
        nb = 1 - pb
        nxt = wid * CPT + (ss + 1) * SB
        if ss + 1 < nstg:
            pltpu.async_copy(row_hbm.at[pl.ds(nxt, SB)], rowv.at[nb], isem_r)
            pltpu.async_copy(col_hbm.at[pl.ds(nxt, SB)], colv.at[nb], isem_c)

        pltpu.async_copy(g_hbm.at[rowv.at[pb, 0]], buf0, gsem0)

        def body(jj, carry2, pb=pb):
            j0 = 2 * jj
            j1 = j0 + 1
            pltpu.async_copy(g_hbm.at[rowv.at[pb, j1]], buf1, gsem1)
            pltpu.make_async_copy(g_hbm.at[rowv.at[pb, j0]], buf0,
                                  gsem0).wait()
            pltpu.sync_copy(buf0, acc.at[colv.at[pb, j0]], add=True)

            @pl.when(jj + 1 < nhalf)
            def _():
                pltpu.async_copy(g_hbm.at[rowv.at[pb, j0 + 2]], buf0, gsem0)

            pltpu.make_async_copy(g_hbm.at[rowv.at[pb, j1]], buf1,
                                  gsem1).wait()
            pltpu.sync_copy(buf1, acc.at[colv.at[pb, j1]], add=True)
            return carry2

        lax.fori_loop(0, nhalf, body, 0)
        if ss + 1 < nstg:
            pltpu.make_async_copy(row_hbm.at[pl.ds(nxt, SB)], rowv.at[nb],
                                  isem_r).wait()
            pltpu.make_async_copy(col_hbm.at[pl.ds(nxt, SB)], colv.at[nb],
                                  isem_c).wait()

    plsc.subcore_barrier()
    for t in range(RPS // CH):
        pltpu.async_copy(acc.at[pl.ds(s * RPS + t * CH, CH)],
                         out_hbm.at[c, pl.ds(s * RPS + t * CH, CH)], zsem)
    for t in range(RPS // CH):
        pltpu.make_async_copy(acc.at[pl.ds(s * RPS + t * CH, CH)],
                              out_hbm.at[c, pl.ds(s * RPS + t * CH, CH)],
                              zsem).wait()


def _prep_body(dp_ref, x_ref, g0_ref, dinv_ref):
    deg = dp_ref[0, :, 0:1] + dp_ref[1, :, 0:1] + 1.0
    dinv = lax.rsqrt(deg)
    dinv_ref[...] = dinv
    g0_ref[...] = x_ref[...] * dinv


_prep_call = pl.pallas_call(
    _prep_body,
    grid=(N // BN,),
    in_specs=[
        pl.BlockSpec((2, BN, D), lambda i: (0, i, 0)),
        pl.BlockSpec((BN, D), lambda i: (i, 0)),
    ],
    out_specs=[
        pl.BlockSpec((BN, D), lambda i: (i, 0)),
        pl.BlockSpec((BN, 1), lambda i: (i, 0)),
    ],
    out_shape=[
        jax.ShapeDtypeStruct((N, D), jnp.float32),
        jax.ShapeDtypeStruct((N, 1), jnp.float32),
    ],
)


def _mid_body(p_ref, g0_ref, dinv_ref, g1_ref):
    d1 = dinv_ref[...]
    g1_ref[...] = (p_ref[0] + p_ref[1] + g0_ref[...]) * (d1 * d1)


_mid_call = pl.pallas_call(
    _mid_body,
    grid=(N // BN,),
    in_specs=[
        pl.BlockSpec((2, BN, D), lambda i: (0, i, 0)),
        pl.BlockSpec((BN, D), lambda i: (i, 0)),
        pl.BlockSpec((BN, 1), lambda i: (i, 0)),
    ],
    out_specs=pl.BlockSpec((BN, D), lambda i: (i, 0)),
    out_shape=jax.ShapeDtypeStruct((N, D), jnp.float32),
)


def _out_body(q_ref, g1_ref, dinv_ref, w_ref, b_ref, o_ref):
    h2 = (q_ref[0] + q_ref[1] + g1_ref[...]) * dinv_ref[...]
    o_ref[...] = jnp.dot(h2, w_ref[...],
                         preferred_element_type=jnp.float32) + b_ref[...]


_out_call = pl.pallas_call(
    _out_body,
    grid=(N // BN,),
    in_specs=[
        pl.BlockSpec((2, BN, D), lambda i: (0, i, 0)),
        pl.BlockSpec((BN, D), lambda i: (i, 0)),
        pl.BlockSpec((BN, 1), lambda i: (i, 0)),
        pl.BlockSpec((D, D_OUT), lambda i: (0, 0)),
        pl.BlockSpec((1, D_OUT), lambda i: (0, 0)),
    ],
    out_specs=pl.BlockSpec((BN, D_OUT), lambda i: (i, 0)),
    out_shape=jax.ShapeDtypeStruct((N, D_OUT), jnp.float32),
)


def kernel(x, edge_index, W, b):
    row = edge_index[0]
    col = edge_index[1]
    pad = EPAD - E
    ar = jnp.arange(pad, dtype=jnp.int32)
    pad_rows = (ar * 37) % N
    pad_cols = N + ar % (NACC - N)
    row2d = jnp.concatenate([row, pad_rows]).reshape(NW * CPT, CH)
    col2d = jnp.concatenate([col, pad_cols]).reshape(NW * CPT, CH)

    ones_rows = jnp.ones((CH, D), jnp.float32)
    z_rows = jnp.zeros((CH, D), jnp.float32)

    dp = _deg_call(col2d, z_rows, ones_rows)
    g0, dinv = _prep_call(dp, x)
    p = _hop_call(row2d, col2d, g0, z_rows)
    g1 = _mid_call(p, g0, dinv)
    q = _hop_call(row2d, col2d, g1, z_rows)
    out = _out_call(q, g1, dinv, W, jnp.reshape(b, (1, D_OUT)))
    return out
